# baseline (speedup 1.0000x reference)
import functools
import math

import jax
import jax.numpy as jnp
from jax import lax
from jax.experimental import pallas as pl
from jax.experimental.pallas import tpu as pltpu
from jax.experimental.pallas import tpu_sc as plsc

D_MODEL = 64
SCALE = math.sqrt(D_MODEL)

NUM_CORES = 2
NUM_SUBCORES = 16
NW = NUM_CORES * NUM_SUBCORES
LANES = 16

STREAM = 128
GROUP_STREAMS = 2
GROUP = STREAM * GROUP_STREAMS
NBUF = 4


def _make_kernel(batch, d):
    assert batch % (NW * GROUP * NBUF) == 0
    b_per_w = batch // NW
    n_groups = b_per_w // GROUP
    n_steps = n_groups // NBUF
    idx_rows = b_per_w // STREAM

    mesh = plsc.VectorSubcoreMesh(core_axis_name="c", subcore_axis_name="s")

    @functools.partial(
        pl.kernel,
        out_type=jax.ShapeDtypeStruct((batch, d), jnp.float32),
        mesh=mesh,
        scratch_types=(
            [pltpu.VMEM((idx_rows, STREAM), jnp.int32)]
            + [pltpu.VMEM((GROUP, d), jnp.float32) for _ in range(NBUF)]
            + [pltpu.SemaphoreType.DMA for _ in range(2 * NBUF)]
        ),
        compiler_params=pltpu.CompilerParams(
            use_tc_tiling_on_sc=False, skip_device_barrier=True
        ),
    )
    def emb_kernel(idx_hbm, lut_hbm, out_hbm, idx_v, *bufs_and_sems):
        bufs = bufs_and_sems[:NBUF]
        gsems = bufs_and_sems[NBUF:2 * NBUF]
        ssems = bufs_and_sems[2 * NBUF:]
        wid = lax.axis_index("s") * NUM_CORES + lax.axis_index("c")
        out_base = wid * b_per_w

        pltpu.sync_copy(idx_hbm.at[pl.ds(wid * idx_rows, idx_rows)], idx_v)

        def fire_gather(g, b):
            for j in range(GROUP_STREAMS):
                pltpu.async_copy(
                    lut_hbm.at[idx_v.at[g * GROUP_STREAMS + j]],
                    bufs[b].at[pl.ds(j * STREAM, STREAM)],
                    gsems[b],
                )

        def wait_gather(b):
            pltpu.make_async_copy(
                lut_hbm.at[pl.ds(0, GROUP)], bufs[b], gsems[b]
            ).wait()

        def scale(b):
            def body(i, c):
                for rr in range(8):
                    r = i * 8 + rr
                    for cc in range(d // LANES):
                        sl = pl.ds(cc * LANES, LANES)
                        bufs[b][r, sl] = bufs[b][r, sl] * SCALE
                return c

            lax.fori_loop(0, GROUP // 8, body, 0, unroll=False)

        def fire_scatter(g, b):
            pltpu.async_copy(
                bufs[b], out_hbm.at[pl.ds(out_base + g * GROUP, GROUP)],
                ssems[b],
            )

        def wait_scatter(b):
            pltpu.make_async_copy(
                bufs[b], out_hbm.at[pl.ds(0, GROUP)], ssems[b]
            ).wait()

        def process(g, b, wait_sct, fire_ahd):
            wait_gather(b)
            scale(b)
            fire_scatter(g, b)
            if fire_ahd:
                bn = (b + NBUF - 1) % NBUF
                if wait_sct:
                    wait_scatter(bn)
                fire_gather(g + NBUF - 1, bn)

        for g in range(NBUF - 1):
            fire_gather(g, g)

        for b in range(NBUF):
            process(b, b, wait_sct=(b != 0), fire_ahd=True)

        def step(s, carry):
            for b in range(NBUF):
                process(s * NBUF + b, b, wait_sct=True, fire_ahd=True)
            return carry

        lax.fori_loop(1, n_steps - 1, step, 0, unroll=False)

        g0 = (n_steps - 1) * NBUF
        process(g0, 0, wait_sct=True, fire_ahd=True)
        for b in range(1, NBUF):
            process(g0 + b, b, wait_sct=False, fire_ahd=False)

        for b in range(NBUF):
            wait_scatter(b)

    return emb_kernel


@jax.jit
def kernel(x, lut):
    b0, b1 = x.shape
    batch = b0 * b1
    d = lut.shape[1]
    idx2d = x.reshape(batch // STREAM, STREAM)
    out = _make_kernel(batch, d)(idx2d, lut)
    return out.reshape(b0, b1, d)


# speedup vs baseline: 1.0005x; 1.0005x over previous
import functools
import math

import jax
import jax.numpy as jnp
from jax import lax
from jax.experimental import pallas as pl
from jax.experimental.pallas import tpu as pltpu
from jax.experimental.pallas import tpu_sc as plsc

D_MODEL = 64
SCALE = math.sqrt(D_MODEL)

NUM_CORES = 2
NUM_SUBCORES = 16
NW = NUM_CORES * NUM_SUBCORES
LANES = 16

SPLIT = 128
NBUF = 4


def _make_kernel(n_rows, row_len, d):
    assert n_rows % (NW * NBUF) == 0
    rows_per_w = n_rows // NW
    n_steps = rows_per_w // NBUF
    splits = ((0, SPLIT), (SPLIT, row_len - SPLIT))

    mesh = plsc.VectorSubcoreMesh(core_axis_name="c", subcore_axis_name="s")

    @functools.partial(
        pl.kernel,
        out_type=jax.ShapeDtypeStruct((n_rows, row_len, d), jnp.float32),
        mesh=mesh,
        scratch_types=(
            [pltpu.VMEM((rows_per_w, row_len), jnp.int32)]
            + [pltpu.VMEM((row_len, d), jnp.float32) for _ in range(NBUF)]
            + [pltpu.SemaphoreType.DMA for _ in range(2 * NBUF)]
        ),
        compiler_params=pltpu.CompilerParams(use_tc_tiling_on_sc=False),
    )
    def emb_kernel(idx_hbm, lut_hbm, out_hbm, idx_v, *bufs_and_sems):
        bufs = bufs_and_sems[:NBUF]
        gsems = bufs_and_sems[NBUF:2 * NBUF]
        ssems = bufs_and_sems[2 * NBUF:]
        wid = lax.axis_index("s") * NUM_CORES + lax.axis_index("c")
        row_base = wid * rows_per_w

        pltpu.sync_copy(idx_hbm.at[pl.ds(row_base, rows_per_w)], idx_v)

        def fire_gather(g, b):
            for lo, ln in splits:
                pltpu.async_copy(
                    lut_hbm.at[idx_v.at[g, pl.ds(lo, ln)]],
                    bufs[b].at[pl.ds(lo, ln)],
                    gsems[b],
                )

        def wait_gather(b):
            pltpu.make_async_copy(
                lut_hbm.at[pl.ds(0, row_len)], bufs[b], gsems[b]
            ).wait()

        def scale(b):
            def body(i, c):
                for rr in range(8):
                    r = i * 8 + rr
                    for cc in range(d // LANES):
                        sl = pl.ds(cc * LANES, LANES)
                        bufs[b][r, sl] = bufs[b][r, sl] * SCALE
                return c

            lax.fori_loop(0, row_len // 8, body, 0, unroll=False)

        def fire_scatter(g, b):
            pltpu.async_copy(bufs[b], out_hbm.at[row_base + g], ssems[b])

        def wait_scatter(b):
            pltpu.make_async_copy(bufs[b], out_hbm.at[0], ssems[b]).wait()

        def process(g, b, wait_sct, fire_ahd):
            wait_gather(b)
            scale(b)
            fire_scatter(g, b)
            if fire_ahd:
                bn = (b + NBUF - 1) % NBUF
                if wait_sct:
                    wait_scatter(bn)
                fire_gather(g + NBUF - 1, bn)

        for g in range(NBUF - 1):
            fire_gather(g, g)

        for b in range(NBUF):
            process(b, b, wait_sct=(b != 0), fire_ahd=True)

        def step(s, carry):
            for b in range(NBUF):
                process(s * NBUF + b, b, wait_sct=True, fire_ahd=True)
            return carry

        lax.fori_loop(1, n_steps - 1, step, 0, unroll=False)

        g0 = (n_steps - 1) * NBUF
        process(g0, 0, wait_sct=True, fire_ahd=True)
        for b in range(1, NBUF):
            process(g0 + b, b, wait_sct=False, fire_ahd=False)

        for b in range(NBUF):
            wait_scatter(b)

    return emb_kernel


@jax.jit
def kernel(x, lut):
    b0, b1 = x.shape
    d = lut.shape[1]
    return _make_kernel(b0, b1, d)(x, lut)


# speedup vs baseline: 1.0024x; 1.0019x over previous
import functools
import math

import jax
import jax.numpy as jnp
from jax import lax
from jax.experimental import pallas as pl
from jax.experimental.pallas import tpu as pltpu
from jax.experimental.pallas import tpu_sc as plsc

D_MODEL = 64
SCALE = math.sqrt(D_MODEL)

NUM_CORES = 2
NUM_SUBCORES = 16
NW = NUM_CORES * NUM_SUBCORES
LANES = 16

SPLIT = 128
NBUF = 4


def _make_kernel(n_rows, row_len, d):
    assert n_rows % (NW * NBUF) == 0
    rows_per_w = n_rows // NW
    n_steps = rows_per_w // NBUF
    splits = ((0, SPLIT), (SPLIT, row_len - SPLIT))

    mesh = plsc.VectorSubcoreMesh(core_axis_name="c", subcore_axis_name="s")

    @functools.partial(
        pl.kernel,
        out_type=jax.ShapeDtypeStruct((n_rows, row_len, d), jnp.float32),
        mesh=mesh,
        scratch_types=(
            [pltpu.VMEM((rows_per_w * row_len,), jnp.int32)]
            + [pltpu.VMEM((row_len, d), jnp.float32) for _ in range(NBUF)]
            + [pltpu.SemaphoreType.DMA for _ in range(2 * NBUF)]
        ),
        compiler_params=pltpu.CompilerParams(use_tc_tiling_on_sc=False),
    )
    def emb_kernel(idx_hbm, lut_hbm, out_hbm, idx_v, *bufs_and_sems):
        bufs = bufs_and_sems[:NBUF]
        gsems = bufs_and_sems[NBUF:2 * NBUF]
        ssems = bufs_and_sems[2 * NBUF:]
        wid = lax.axis_index("s") * NUM_CORES + lax.axis_index("c")
        row_base = wid * rows_per_w

        pltpu.sync_copy(
            idx_hbm.at[pl.ds(row_base * row_len, rows_per_w * row_len)], idx_v
        )

        def fire_gather(g, b):
            for lo, ln in splits:
                pltpu.async_copy(
                    lut_hbm.at[idx_v.at[pl.ds(g * row_len + lo, ln)]],
                    bufs[b].at[pl.ds(lo, ln)],
                    gsems[b],
                )

        def wait_gather(b):
            pltpu.make_async_copy(
                lut_hbm.at[pl.ds(0, row_len)], bufs[b], gsems[b]
            ).wait()

        def scale(b):
            def body(i, c):
                for rr in range(8):
                    r = i * 8 + rr
                    for cc in range(d // LANES):
                        sl = pl.ds(cc * LANES, LANES)
                        bufs[b][r, sl] = bufs[b][r, sl] * SCALE
                return c

            lax.fori_loop(0, row_len // 8, body, 0, unroll=False)

        def fire_scatter(g, b):
            pltpu.async_copy(bufs[b], out_hbm.at[row_base + g], ssems[b])

        def wait_scatter(b):
            pltpu.make_async_copy(bufs[b], out_hbm.at[0], ssems[b]).wait()

        def process(g, b, wait_sct, fire_ahd):
            wait_gather(b)
            scale(b)
            fire_scatter(g, b)
            if fire_ahd:
                bn = (b + NBUF - 1) % NBUF
                if wait_sct:
                    wait_scatter(bn)
                fire_gather(g + NBUF - 1, bn)

        for g in range(NBUF - 1):
            fire_gather(g, g)

        for b in range(NBUF):
            process(b, b, wait_sct=(b != 0), fire_ahd=True)

        def step(s, carry):
            for b in range(NBUF):
                process(s * NBUF + b, b, wait_sct=True, fire_ahd=True)
            return carry

        lax.fori_loop(1, n_steps - 1, step, 0, unroll=False)

        g0 = (n_steps - 1) * NBUF
        process(g0, 0, wait_sct=True, fire_ahd=True)
        for b in range(1, NBUF):
            process(g0 + b, b, wait_sct=False, fire_ahd=False)

        for b in range(NBUF):
            wait_scatter(b)

    return emb_kernel


@jax.jit
def kernel(x, lut):
    b0, b1 = x.shape
    d = lut.shape[1]
    return _make_kernel(b0, b1, d)(x.reshape(-1), lut)
